# Initial kernel scaffold; baseline (speedup 1.0000x reference)
import jax
import jax.numpy as jnp
from jax.experimental import pallas as pl


def kernel(input_ids, table):
    raise NotImplementedError("write your pallas kernel here")


# speedup vs baseline: 1.2973x; 1.2973x over previous
import functools

import jax
import jax.numpy as jnp
from jax import lax
from jax.experimental import pallas as pl
from jax.experimental.pallas import tpu as pltpu
from jax.experimental.pallas import tpu_sc as plsc

_NC = 2
_NS = 16
_NW = _NC * _NS

_B = 1024 * 77
_D = 768
_BPW = _B // _NW
_CH = 56
_NCHUNK = _BPW // _CH


def _make_gather():
    mesh = plsc.VectorSubcoreMesh(
        core_axis_name="c", subcore_axis_name="s",
        num_cores=_NC, num_subcores=_NS)

    @functools.partial(
        pl.kernel,
        mesh=mesh,
        out_type=jax.ShapeDtypeStruct((_B, _D), jnp.float32),
        scratch_types=[
            pltpu.VMEM((_BPW,), jnp.int32),
            pltpu.VMEM((2, _CH, _D), jnp.float32),
            pltpu.SemaphoreType.DMA,
            pltpu.SemaphoreType.DMA,
            pltpu.SemaphoreType.DMA,
            pltpu.SemaphoreType.DMA,
        ],
    )
    def gather_kernel(idx_hbm, table_hbm, out_hbm, idx_v, rows_v,
                      gsem0, gsem1, osem0, osem1):
        wid = lax.axis_index("s") * _NC + lax.axis_index("c")
        base = wid * _BPW
        pltpu.sync_copy(idx_hbm.at[pl.ds(base, _BPW)], idx_v)

        gsems = (gsem0, gsem1)
        osems = (osem0, osem1)

        def g_start(c, slot):
            pltpu.async_copy(table_hbm.at[idx_v.at[pl.ds(c * _CH, _CH)]], rows_v.at[slot],
                             gsems[slot])

        def g_wait(c, slot):
            pltpu.make_async_copy(table_hbm.at[idx_v.at[pl.ds(c * _CH, _CH)]], rows_v.at[slot],
                                  gsems[slot]).wait()

        def o_start(c, slot):
            pltpu.async_copy(rows_v.at[slot],
                             out_hbm.at[pl.ds(base + c * _CH, _CH)],
                             osems[slot])

        def o_wait(c, slot):
            pltpu.make_async_copy(rows_v.at[slot],
                                  out_hbm.at[pl.ds(base + c * _CH, _CH)],
                                  osems[slot]).wait()

        g_start(0, 0)
        g_start(1, 1)

        def body(p, _):
            for slot in range(2):
                c = 2 * p + slot
                g_wait(c, slot)
                o_start(c, slot)

                @pl.when(c + 2 < _NCHUNK)
                def _():
                    o_wait(c, slot)
                    g_start(c + 2, slot)
            return 0

        lax.fori_loop(0, _NCHUNK // 2, body, 0)
        for slot, c in ((0, _NCHUNK - 2), (1, _NCHUNK - 1)):
            o_wait(c, slot)

    return gather_kernel


_gather = _make_gather()


@jax.jit
def kernel(input_ids, table):
    ids_flat = input_ids.reshape(_B)
    out = _gather(ids_flat, table)
    return out.reshape(input_ids.shape + (table.shape[1],))
